# Initial kernel scaffold; baseline (speedup 1.0000x reference)
import functools

import jax
import jax.numpy as jnp
from jax import lax
from jax.experimental import pallas as pl
from jax.experimental.pallas import tpu as pltpu
from jax.experimental.pallas import tpu_sc as plsc

N = 10000
E = 320000
D = 128
CHUNK = 80
NCHUNKS = E // CHUNK
NCORES = 2
NSUB = 16
NW = NCORES * NSUB
NB = N // NSUB
ZR = 125

_mesh = lambda: plsc.VectorSubcoreMesh(core_axis_name="c", subcore_axis_name="s")


def _make_segsum(split_planes: bool):
    nct = (NCHUNKS // NSUB) if split_planes else (NCHUNKS // NW)

    @functools.partial(
        pl.kernel,
        out_type=jax.ShapeDtypeStruct((2 * N, D), jnp.float32),
        mesh=_mesh(),
        scratch_types=[
            pltpu.VMEM_SHARED((N, D), jnp.float32),
            pltpu.VMEM((nct, CHUNK), jnp.int32),
            pltpu.VMEM((nct, CHUNK), jnp.int32),
            pltpu.VMEM((CHUNK, D), jnp.float32),
            pltpu.VMEM((CHUNK, D), jnp.float32),
            pltpu.VMEM((ZR, D), jnp.float32),
            pltpu.SemaphoreType.DMA,
            pltpu.SemaphoreType.DMA,
        ],
    )
    def segsum(h_hbm, rowc_hbm, colc_hbm, out_hbm,
               acc, rowbuf, colbuf, rows0, rows1, zbuf, sem0, sem1):
        cid = lax.axis_index("c")
        sid = lax.axis_index("s")
        if split_planes:
            rbase = sid * nct
            cbase = cid * NCHUNKS + sid * nct
        else:
            rbase = (cid * NSUB + sid) * nct
            cbase = rbase
        pltpu.sync_copy(rowc_hbm.at[pl.ds(rbase, nct)], rowbuf)
        pltpu.sync_copy(colc_hbm.at[pl.ds(cbase, nct)], colbuf)

        zv = jnp.zeros((16,), jnp.float32)

        def zrow(i, _):
            def zlane(k, c):
                zbuf[i, pl.ds(k * 16, 16)] = zv
                return c
            return lax.fori_loop(0, D // 16, zlane, 0)

        lax.fori_loop(0, ZR, zrow, 0)

        def zslab(k, _):
            pltpu.sync_copy(zbuf, acc.at[pl.ds(sid * NB + k * ZR, ZR)])
            return 0

        lax.fori_loop(0, NB // ZR, zslab, 0)
        plsc.subcore_barrier()

        def step(p, _):
            j0 = 2 * p
            j1 = j0 + 1
            g0 = pltpu.async_copy(h_hbm.at[colbuf.at[j0]], rows0, sem0)
            g1 = pltpu.async_copy(h_hbm.at[colbuf.at[j1]], rows1, sem1)
            g0.wait()
            pltpu.sync_copy(rows0, acc.at[rowbuf.at[j0]], add=True)
            g1.wait()
            pltpu.sync_copy(rows1, acc.at[rowbuf.at[j1]], add=True)
            return 0

        lax.fori_loop(0, nct // 2, step, 0)
        plsc.subcore_barrier()

        pltpu.sync_copy(acc.at[pl.ds(sid * NB, NB)],
                        out_hbm.at[pl.ds(cid * N + sid * NB, NB)])

    return segsum


def _make_deg():
    DW = 16
    nct = NCHUNKS // NW

    @functools.partial(
        pl.kernel,
        out_type=jax.ShapeDtypeStruct((2 * N, DW), jnp.float32),
        mesh=_mesh(),
        scratch_types=[
            pltpu.VMEM_SHARED((N, DW), jnp.float32),
            pltpu.VMEM((nct, CHUNK), jnp.int32),
            pltpu.VMEM((CHUNK, DW), jnp.float32),
            pltpu.VMEM((ZR, DW), jnp.float32),
        ],
    )
    def deg(rowc_hbm, out_hbm, acc, rowbuf, ones, zbuf):
        cid = lax.axis_index("c")
        sid = lax.axis_index("s")
        base = (cid * NSUB + sid) * nct
        pltpu.sync_copy(rowc_hbm.at[pl.ds(base, nct)], rowbuf)

        lanes = lax.iota(jnp.int32, 16)
        onev = jnp.where(lanes == 0, 1.0, 0.0).astype(jnp.float32)
        zv = jnp.zeros((16,), jnp.float32)

        def fill(i, _):
            ones[i] = onev
            return 0

        lax.fori_loop(0, CHUNK, fill, 0)

        def zrow(i, _):
            zbuf[i] = zv
            return 0

        lax.fori_loop(0, ZR, zrow, 0)

        def zslab(k, _):
            pltpu.sync_copy(zbuf, acc.at[pl.ds(sid * NB + k * ZR, ZR)])
            return 0

        lax.fori_loop(0, NB // ZR, zslab, 0)
        plsc.subcore_barrier()

        def step(j, _):
            pltpu.sync_copy(ones, acc.at[rowbuf.at[j]], add=True)
            return 0

        lax.fori_loop(0, nct, step, 0)
        plsc.subcore_barrier()

        pltpu.sync_copy(acc.at[pl.ds(sid * NB, NB)],
                        out_hbm.at[pl.ds(cid * N + sid * NB, NB)])

    return deg


_seg_edges = _make_segsum(False)
_seg_planes = _make_segsum(True)
_deg = _make_deg()


BM = 2000
GRID = (N // BM,)


def _b2(d):
    return pl.BlockSpec((BM, d), lambda i: (i, 0))


def _b3(d):
    return pl.BlockSpec((2, BM, d), lambda i: (0, i, 0))


def _bfull(*shape):
    return pl.BlockSpec(shape, lambda i: tuple(0 for _ in shape))


def _dot(a, b):
    return jnp.dot(a, b, preferred_element_type=jnp.float32)


def _tc1_body(deg2, x, xs, dis, invc):
    d = deg2[0, :, 0:1] + deg2[1, :, 0:1]
    pos = d > 0.0
    di = jnp.where(pos, lax.rsqrt(d), 0.0)
    dis[...] = di
    invc[...] = jnp.where(pos, 1.0 / d, 0.0)
    xs[...] = x[...] * di


def _tc2_body(x, s0, dis, W0, b0, al, x1):
    s = (s0[0] + s0[1]) * dis[...]
    t = _dot(s, W0[...]) + b0[...]
    a0 = al[...][:, 0:1]
    x1[...] = x[...] + a0 * t


def _tc3_body(s1, invc, x1, W1, Wr1, b1, x2p):
    agg = (s1[0] + s1[1]) * invc[...]
    r = _dot(agg, W1[...]) + _dot(x1[...], Wr1[...]) + b1[...]
    r = jnp.maximum(r, 0.0)
    x2p[0] = r[:, :128]
    x2p[1] = r[:, 128:]


def _tc4_body(s2, invc, dis, x2p, W2, Wr2, b2, x3p, x3sp):
    ic = invc[...]
    W2v = W2[...]
    Wr2v = Wr2[...]
    r = (_dot(s2[0] * ic, W2v[:128, :]) + _dot(s2[1] * ic, W2v[128:, :])
         + _dot(x2p[0], Wr2v[:128, :]) + _dot(x2p[1], Wr2v[128:, :])
         + b2[...])
    r = jnp.maximum(r, 0.0)
    di = dis[...]
    x3p[0] = r[:, :128]
    x3p[1] = r[:, 128:]
    x3sp[0] = r[:, :128] * di
    x3sp[1] = r[:, 128:] * di


def _tc5_body(s3, dis, x3p, W3, b3, W4p, Wr4, b4, al, h4, r4):
    di = dis[...]
    W3v = W3[...]
    t = _dot(s3[0] * di, W3v[:128, :]) + _dot(s3[1] * di, W3v[128:, :]) + b3[...]
    a3 = al[...][:, 3:4]
    x4_0 = x3p[0] + a3 * t[:, :128]
    x4_1 = x3p[1] + a3 * t[:, 128:]
    W4v = W4p[...]
    Wr4v = Wr4[...]
    h4[...] = _dot(x4_0, W4v[:128, :]) + _dot(x4_1, W4v[128:, :])
    r4[...] = _dot(x4_0, Wr4v[:128, :]) + _dot(x4_1, Wr4v[128:, :]) + b4[...]


def _tc6_body(s4, invc, r4, x5):
    agg = (s4[0] + s4[1])[:, :112] * invc[...]
    x5[...] = agg + r4[...]


def _f32(*shape):
    return jax.ShapeDtypeStruct(shape, jnp.float32)


def kernel(x, edge_index, alpha, W0, b0, W1, Wr1, b1, W2, Wr2, b2,
           W3, b3, W4, Wr4, b4):
    row = edge_index[0].astype(jnp.int32).reshape(NCHUNKS, CHUNK)
    col = edge_index[1].astype(jnp.int32).reshape(NCHUNKS, CHUNK)
    colp = jnp.concatenate([col, col + N], axis=0)
    al = alpha.reshape(1, 5)
    b0r = b0.reshape(1, 128)
    b1r = b1.reshape(1, 256)
    b2r = b2.reshape(1, 256)
    b3r = b3.reshape(1, 256)
    b4r = b4.reshape(1, 112)
    W4p = jnp.pad(W4, ((0, 0), (0, 16)))

    deg2 = _deg(row).reshape(2, N, 16)

    xs, dis, invc = pl.pallas_call(
        _tc1_body, grid=GRID,
        in_specs=[_b3(16), _b2(128)],
        out_specs=[_b2(128), _b2(1), _b2(1)],
        out_shape=[_f32(N, 128), _f32(N, 1), _f32(N, 1)],
    )(deg2, x)

    s0 = _seg_edges(xs, row, col).reshape(2, N, 128)

    x1 = pl.pallas_call(
        _tc2_body, grid=GRID,
        in_specs=[_b2(128), _b3(128), _b2(1), _bfull(128, 128),
                  _bfull(1, 128), _bfull(1, 5)],
        out_specs=_b2(128),
        out_shape=_f32(N, 128),
    )(x, s0, dis, W0, b0r, al)

    s1 = _seg_edges(x1, row, col).reshape(2, N, 128)

    x2p = pl.pallas_call(
        _tc3_body, grid=GRID,
        in_specs=[_b3(128), _b2(1), _b2(128), _bfull(128, 256),
                  _bfull(128, 256), _bfull(1, 256)],
        out_specs=_b3(128),
        out_shape=_f32(2, N, 128),
    )(s1, invc, x1, W1, Wr1, b1r)

    s2 = _seg_planes(x2p.reshape(2 * N, 128), row, colp).reshape(2, N, 128)

    x3p, x3sp = pl.pallas_call(
        _tc4_body, grid=GRID,
        in_specs=[_b3(128), _b2(1), _b2(1), _b3(128), _bfull(256, 256),
                  _bfull(256, 256), _bfull(1, 256)],
        out_specs=[_b3(128), _b3(128)],
        out_shape=[_f32(2, N, 128), _f32(2, N, 128)],
    )(s2, invc, dis, x2p, W2, Wr2, b2r)

    s3 = _seg_planes(x3sp.reshape(2 * N, 128), row, colp).reshape(2, N, 128)

    h4, r4 = pl.pallas_call(
        _tc5_body, grid=GRID,
        in_specs=[_b3(128), _b2(1), _b3(128), _bfull(256, 256),
                  _bfull(1, 256), _bfull(256, 128), _bfull(256, 112),
                  _bfull(1, 112), _bfull(1, 5)],
        out_specs=[_b2(128), _b2(112)],
        out_shape=[_f32(N, 128), _f32(N, 112)],
    )(s3, dis, x3p, W3, b3r, W4p, Wr4, b4r, al)

    s4 = _seg_edges(h4, row, col).reshape(2, N, 128)

    x5 = pl.pallas_call(
        _tc6_body, grid=GRID,
        in_specs=[_b3(128), _b2(1), _b2(112)],
        out_specs=_b2(112),
        out_shape=_f32(N, 112),
    )(s4, invc, r4)

    return x5


# speedup vs baseline: 4.2134x; 4.2134x over previous
import functools

import jax
import jax.numpy as jnp
from jax import lax
from jax.experimental import pallas as pl
from jax.experimental.pallas import tpu as pltpu
from jax.experimental.pallas import tpu_sc as plsc

N = 10000
E = 320000
D = 128
CHUNK = 128
EPAD = 327680
NCHUNKS = EPAD // CHUNK
NCORES = 2
NSUB = 16
NW = NCORES * NSUB
ACC_R = 10240
DUMMY = N
NB = ACC_R // NSUB
IB = 16

_mesh = lambda: plsc.VectorSubcoreMesh(core_axis_name="c", subcore_axis_name="s")


def _make_segsum(split_planes: bool):
    nct = (NCHUNKS // NSUB) if split_planes else (NCHUNKS // NW)

    @functools.partial(
        pl.kernel,
        out_type=jax.ShapeDtypeStruct((2 * ACC_R, D), jnp.float32),
        mesh=_mesh(),
        scratch_types=[
            pltpu.VMEM_SHARED((ACC_R, D), jnp.float32),
            pltpu.VMEM((IB, CHUNK), jnp.int32),
            pltpu.VMEM((IB, CHUNK), jnp.int32),
            pltpu.VMEM((CHUNK, D), jnp.float32),
            pltpu.VMEM((CHUNK, D), jnp.float32),
            pltpu.SemaphoreType.DMA,
            pltpu.SemaphoreType.DMA,
        ],
    )
    def segsum(h_hbm, rowc_hbm, colc_hbm, zeros_hbm, out_hbm,
               acc, rowbuf, colbuf, rows0, rows1, sem0, sem1):
        cid = lax.axis_index("c")
        sid = lax.axis_index("s")
        if split_planes:
            rbase = sid * nct
            cbase = cid * NCHUNKS + sid * nct
        else:
            rbase = (cid * NSUB + sid) * nct
            cbase = rbase

        pltpu.sync_copy(zeros_hbm, acc.at[pl.ds(sid * NB, NB)])
        plsc.subcore_barrier()

        def blk(b, _):
            pltpu.sync_copy(rowc_hbm.at[pl.ds(rbase + b * IB, IB)], rowbuf)
            pltpu.sync_copy(colc_hbm.at[pl.ds(cbase + b * IB, IB)], colbuf)

            def step(p, _):
                j0 = 2 * p
                j1 = j0 + 1
                g0 = pltpu.async_copy(h_hbm.at[colbuf.at[j0]], rows0, sem0)
                g1 = pltpu.async_copy(h_hbm.at[colbuf.at[j1]], rows1, sem1)
                g0.wait()
                pltpu.sync_copy(rows0, acc.at[rowbuf.at[j0]], add=True)
                g1.wait()
                pltpu.sync_copy(rows1, acc.at[rowbuf.at[j1]], add=True)
                return 0

            lax.fori_loop(0, IB // 2, step, 0)
            return 0

        lax.fori_loop(0, nct // IB, blk, 0)
        plsc.subcore_barrier()

        pltpu.sync_copy(acc.at[pl.ds(sid * NB, NB)],
                        out_hbm.at[pl.ds(cid * ACC_R + sid * NB, NB)])

    return segsum


def _make_deg():
    nct = NCHUNKS // NW

    @functools.partial(
        pl.kernel,
        out_type=jax.ShapeDtypeStruct((2 * ACC_R, D), jnp.float32),
        mesh=_mesh(),
        scratch_types=[
            pltpu.VMEM_SHARED((ACC_R, D), jnp.float32),
            pltpu.VMEM((nct, CHUNK), jnp.int32),
            pltpu.VMEM((CHUNK, D), jnp.float32),
        ],
    )
    def deg(rowc_hbm, ones_hbm, zeros_hbm, out_hbm, acc, rowbuf, ones):
        cid = lax.axis_index("c")
        sid = lax.axis_index("s")
        base = (cid * NSUB + sid) * nct
        pltpu.sync_copy(rowc_hbm.at[pl.ds(base, nct)], rowbuf)
        pltpu.sync_copy(ones_hbm, ones)
        pltpu.sync_copy(zeros_hbm, acc.at[pl.ds(sid * NB, NB)])
        plsc.subcore_barrier()

        def step(j, _):
            pltpu.sync_copy(ones, acc.at[rowbuf.at[j]], add=True)
            return 0

        lax.fori_loop(0, nct, step, 0)
        plsc.subcore_barrier()

        pltpu.sync_copy(acc.at[pl.ds(sid * NB, NB)],
                        out_hbm.at[pl.ds(cid * ACC_R + sid * NB, NB)])

    return deg


_seg_edges = _make_segsum(False)
_seg_planes = _make_segsum(True)
_deg = _make_deg()


BM = 2000
GRID = (N // BM,)


def _b2(d):
    return pl.BlockSpec((BM, d), lambda i: (i, 0))


def _b3(d):
    return pl.BlockSpec((2, BM, d), lambda i: (0, i, 0))


def _bfull(*shape):
    return pl.BlockSpec(shape, lambda i: tuple(0 for _ in shape))


def _dot(a, b):
    return jnp.dot(a, b, preferred_element_type=jnp.float32)


def _tc1_body(deg2, x, xs, dis, invc):
    d = deg2[0, :, 0:1] + deg2[1, :, 0:1]
    pos = d > 0.0
    di = jnp.where(pos, lax.rsqrt(d), 0.0)
    dis[...] = di
    invc[...] = jnp.where(pos, 1.0 / d, 0.0)
    xs[...] = x[...] * di


def _tc2_body(x, s0, dis, W0, b0, al, x1):
    s = (s0[0] + s0[1]) * dis[...]
    t = _dot(s, W0[...]) + b0[...]
    a0 = al[...][:, 0:1]
    x1[...] = x[...] + a0 * t


def _tc3_body(s1, invc, x1, W1, Wr1, b1, x2p):
    agg = (s1[0] + s1[1]) * invc[...]
    r = _dot(agg, W1[...]) + _dot(x1[...], Wr1[...]) + b1[...]
    r = jnp.maximum(r, 0.0)
    x2p[0] = r[:, :128]
    x2p[1] = r[:, 128:]


def _tc4_body(s2, invc, dis, x2p, W2, Wr2, b2, x3p, x3sp):
    ic = invc[...]
    W2v = W2[...]
    Wr2v = Wr2[...]
    r = (_dot(s2[0] * ic, W2v[:128, :]) + _dot(s2[1] * ic, W2v[128:, :])
         + _dot(x2p[0], Wr2v[:128, :]) + _dot(x2p[1], Wr2v[128:, :])
         + b2[...])
    r = jnp.maximum(r, 0.0)
    di = dis[...]
    x3p[0] = r[:, :128]
    x3p[1] = r[:, 128:]
    x3sp[0] = r[:, :128] * di
    x3sp[1] = r[:, 128:] * di


def _tc5_body(s3, dis, x3p, W3, b3, W4p, Wr4, b4, al, h4, r4):
    di = dis[...]
    W3v = W3[...]
    t = _dot(s3[0] * di, W3v[:128, :]) + _dot(s3[1] * di, W3v[128:, :]) + b3[...]
    a3 = al[...][:, 3:4]
    x4_0 = x3p[0] + a3 * t[:, :128]
    x4_1 = x3p[1] + a3 * t[:, 128:]
    W4v = W4p[...]
    Wr4v = Wr4[...]
    h4[...] = _dot(x4_0, W4v[:128, :]) + _dot(x4_1, W4v[128:, :])
    r4[...] = _dot(x4_0, Wr4v[:128, :]) + _dot(x4_1, Wr4v[128:, :]) + b4[...]


def _tc6_body(s4, invc, r4, x5):
    agg = (s4[0] + s4[1])[:, :112] * invc[...]
    x5[...] = agg + r4[...]


def _f32(*shape):
    return jax.ShapeDtypeStruct(shape, jnp.float32)


def kernel(x, edge_index, alpha, W0, b0, W1, Wr1, b1, W2, Wr2, b2,
           W3, b3, W4, Wr4, b4):
    npad = EPAD - E
    row = jnp.concatenate(
        [edge_index[0].astype(jnp.int32),
         jnp.full((npad,), DUMMY, jnp.int32)]).reshape(NCHUNKS, CHUNK)
    col = jnp.concatenate(
        [edge_index[1].astype(jnp.int32),
         jnp.zeros((npad,), jnp.int32)]).reshape(NCHUNKS, CHUNK)
    colp = jnp.concatenate([col, col + N], axis=0)
    zeros_d = jnp.zeros((NB, D), jnp.float32)
    ones_d = jnp.ones((CHUNK, D), jnp.float32)
    al = alpha.reshape(1, 5)
    b0r = b0.reshape(1, 128)
    b1r = b1.reshape(1, 256)
    b2r = b2.reshape(1, 256)
    b3r = b3.reshape(1, 256)
    b4r = b4.reshape(1, 112)
    W4p = jnp.pad(W4, ((0, 0), (0, 16)))

    deg2 = _deg(row, ones_d, zeros_d).reshape(2, ACC_R, 128)

    xs, dis, invc = pl.pallas_call(
        _tc1_body, grid=GRID,
        in_specs=[_b3(128), _b2(128)],
        out_specs=[_b2(128), _b2(1), _b2(1)],
        out_shape=[_f32(N, 128), _f32(N, 1), _f32(N, 1)],
    )(deg2, x)

    s0 = _seg_edges(xs, row, col, zeros_d).reshape(2, ACC_R, 128)

    x1 = pl.pallas_call(
        _tc2_body, grid=GRID,
        in_specs=[_b2(128), _b3(128), _b2(1), _bfull(128, 128),
                  _bfull(1, 128), _bfull(1, 5)],
        out_specs=_b2(128),
        out_shape=_f32(N, 128),
    )(x, s0, dis, W0, b0r, al)

    s1 = _seg_edges(x1, row, col, zeros_d).reshape(2, ACC_R, 128)

    x2p = pl.pallas_call(
        _tc3_body, grid=GRID,
        in_specs=[_b3(128), _b2(1), _b2(128), _bfull(128, 256),
                  _bfull(128, 256), _bfull(1, 256)],
        out_specs=_b3(128),
        out_shape=_f32(2, N, 128),
    )(s1, invc, x1, W1, Wr1, b1r)

    s2 = _seg_planes(x2p.reshape(2 * N, 128), row, colp,
                     zeros_d).reshape(2, ACC_R, 128)

    x3p, x3sp = pl.pallas_call(
        _tc4_body, grid=GRID,
        in_specs=[_b3(128), _b2(1), _b2(1), _b3(128), _bfull(256, 256),
                  _bfull(256, 256), _bfull(1, 256)],
        out_specs=[_b3(128), _b3(128)],
        out_shape=[_f32(2, N, 128), _f32(2, N, 128)],
    )(s2, invc, dis, x2p, W2, Wr2, b2r)

    s3 = _seg_planes(x3sp.reshape(2 * N, 128), row, colp,
                     zeros_d).reshape(2, ACC_R, 128)

    h4, r4 = pl.pallas_call(
        _tc5_body, grid=GRID,
        in_specs=[_b3(128), _b2(1), _b3(128), _bfull(256, 256),
                  _bfull(1, 256), _bfull(256, 128), _bfull(256, 112),
                  _bfull(1, 112), _bfull(1, 5)],
        out_specs=[_b2(128), _b2(112)],
        out_shape=[_f32(N, 128), _f32(N, 112)],
    )(s3, dis, x3p, W3, b3r, W4p, Wr4, b4r, al)

    s4 = _seg_edges(h4, row, col, zeros_d).reshape(2, ACC_R, 128)

    x5 = pl.pallas_call(
        _tc6_body, grid=GRID,
        in_specs=[_b3(128), _b2(1), _b2(112)],
        out_specs=_b2(112),
        out_shape=_f32(N, 112),
    )(s4, invc, r4)

    return x5


# speedup vs baseline: 4.3467x; 1.0316x over previous
import functools

import jax
import jax.numpy as jnp
from jax import lax
from jax.experimental import pallas as pl
from jax.experimental.pallas import tpu as pltpu
from jax.experimental.pallas import tpu_sc as plsc

N = 10000
E = 320000
D = 128
CHUNK = 64
EPAD = 327680
NCHUNKS = EPAD // CHUNK
NCORES = 2
NSUB = 16
NW = NCORES * NSUB
ACC_R = 10240
DUMMY = N
NB = ACC_R // NSUB
IB = 16

_mesh = lambda: plsc.VectorSubcoreMesh(core_axis_name="c", subcore_axis_name="s")


def _make_segsum(split_planes: bool):
    nct = (NCHUNKS // NSUB) if split_planes else (NCHUNKS // NW)
    NBUF = 4
    LAG = 2

    @functools.partial(
        pl.kernel,
        out_type=jax.ShapeDtypeStruct((2 * ACC_R, D), jnp.float32),
        mesh=_mesh(),
        scratch_types=[
            pltpu.VMEM_SHARED((ACC_R, D), jnp.float32),
            pltpu.VMEM((IB, CHUNK), jnp.int32),
            pltpu.VMEM((IB, CHUNK), jnp.int32),
            [pltpu.VMEM((CHUNK, D), jnp.float32) for _ in range(NBUF)],
            [pltpu.SemaphoreType.DMA for _ in range(NBUF)],
            [pltpu.SemaphoreType.DMA for _ in range(NBUF)],
        ],
    )
    def segsum(h_hbm, rowc_hbm, colc_hbm, zeros_hbm, out_hbm,
               acc, rowbuf, colbuf, bufs, gsems, ssems):
        cid = lax.axis_index("c")
        sid = lax.axis_index("s")
        if split_planes:
            rbase = sid * nct
            cbase = cid * NCHUNKS + sid * nct
        else:
            rbase = (cid * NSUB + sid) * nct
            cbase = rbase

        pltpu.sync_copy(zeros_hbm, acc.at[pl.ds(sid * NB, NB)])
        plsc.subcore_barrier()

        def blk(b, _):
            pltpu.sync_copy(rowc_hbm.at[pl.ds(rbase + b * IB, IB)], rowbuf)
            pltpu.sync_copy(colc_hbm.at[pl.ds(cbase + b * IB, IB)], colbuf)

            gd = [None] * IB
            sd = [None] * IB
            for j in range(IB):
                r = j % NBUF
                if j >= NBUF:
                    sd[j - NBUF].wait()
                gd[j] = pltpu.async_copy(h_hbm.at[colbuf.at[j]], bufs[r],
                                         gsems[r])
                if j >= LAG:
                    k = j - LAG
                    gd[k].wait()
                    sd[k] = pltpu.async_copy(bufs[k % NBUF],
                                             acc.at[rowbuf.at[k]],
                                             ssems[k % NBUF], add=True)
            for k in range(IB - LAG, IB):
                gd[k].wait()
                sd[k] = pltpu.async_copy(bufs[k % NBUF],
                                         acc.at[rowbuf.at[k]],
                                         ssems[k % NBUF], add=True)
            for k in range(IB - NBUF, IB):
                sd[k].wait()
            return 0

        lax.fori_loop(0, nct // IB, blk, 0)
        plsc.subcore_barrier()

        pltpu.sync_copy(acc.at[pl.ds(sid * NB, NB)],
                        out_hbm.at[pl.ds(cid * ACC_R + sid * NB, NB)])

    return segsum


def _make_deg():
    nct = NCHUNKS // NW

    @functools.partial(
        pl.kernel,
        out_type=jax.ShapeDtypeStruct((2 * ACC_R, D), jnp.float32),
        mesh=_mesh(),
        scratch_types=[
            pltpu.VMEM_SHARED((ACC_R, D), jnp.float32),
            pltpu.VMEM((nct, CHUNK), jnp.int32),
            pltpu.VMEM((CHUNK, D), jnp.float32),
        ],
    )
    def deg(rowc_hbm, ones_hbm, zeros_hbm, out_hbm, acc, rowbuf, ones):
        cid = lax.axis_index("c")
        sid = lax.axis_index("s")
        base = (cid * NSUB + sid) * nct
        pltpu.sync_copy(rowc_hbm.at[pl.ds(base, nct)], rowbuf)
        pltpu.sync_copy(ones_hbm, ones)
        pltpu.sync_copy(zeros_hbm, acc.at[pl.ds(sid * NB, NB)])
        plsc.subcore_barrier()

        def step(j, _):
            pltpu.sync_copy(ones, acc.at[rowbuf.at[j]], add=True)
            return 0

        lax.fori_loop(0, nct, step, 0)
        plsc.subcore_barrier()

        pltpu.sync_copy(acc.at[pl.ds(sid * NB, NB)],
                        out_hbm.at[pl.ds(cid * ACC_R + sid * NB, NB)])

    return deg


_seg_edges = _make_segsum(False)
_seg_planes = _make_segsum(True)
_deg = _make_deg()


BM = 2000
GRID = (N // BM,)


def _b2(d):
    return pl.BlockSpec((BM, d), lambda i: (i, 0))


def _b3(d):
    return pl.BlockSpec((2, BM, d), lambda i: (0, i, 0))


def _bfull(*shape):
    return pl.BlockSpec(shape, lambda i: tuple(0 for _ in shape))


def _dot(a, b):
    return jnp.dot(a, b, preferred_element_type=jnp.float32)


def _tc1_body(deg2, x, xs, dis, invc):
    d = deg2[0, :, 0:1] + deg2[1, :, 0:1]
    pos = d > 0.0
    di = jnp.where(pos, lax.rsqrt(d), 0.0)
    dis[...] = di
    invc[...] = jnp.where(pos, 1.0 / d, 0.0)
    xs[...] = x[...] * di


def _tc2_body(x, s0, dis, W0, b0, al, x1):
    s = (s0[0] + s0[1]) * dis[...]
    t = _dot(s, W0[...]) + b0[...]
    a0 = al[...][:, 0:1]
    x1[...] = x[...] + a0 * t


def _tc3_body(s1, invc, x1, W1, Wr1, b1, x2p):
    agg = (s1[0] + s1[1]) * invc[...]
    r = _dot(agg, W1[...]) + _dot(x1[...], Wr1[...]) + b1[...]
    r = jnp.maximum(r, 0.0)
    x2p[0] = r[:, :128]
    x2p[1] = r[:, 128:]


def _tc4_body(s2, invc, dis, x2p, W2, Wr2, b2, x3p, x3sp):
    ic = invc[...]
    W2v = W2[...]
    Wr2v = Wr2[...]
    r = (_dot(s2[0] * ic, W2v[:128, :]) + _dot(s2[1] * ic, W2v[128:, :])
         + _dot(x2p[0], Wr2v[:128, :]) + _dot(x2p[1], Wr2v[128:, :])
         + b2[...])
    r = jnp.maximum(r, 0.0)
    di = dis[...]
    x3p[0] = r[:, :128]
    x3p[1] = r[:, 128:]
    x3sp[0] = r[:, :128] * di
    x3sp[1] = r[:, 128:] * di


def _tc5_body(s3, dis, x3p, W3, b3, W4p, Wr4, b4, al, h4, r4):
    di = dis[...]
    W3v = W3[...]
    t = _dot(s3[0] * di, W3v[:128, :]) + _dot(s3[1] * di, W3v[128:, :]) + b3[...]
    a3 = al[...][:, 3:4]
    x4_0 = x3p[0] + a3 * t[:, :128]
    x4_1 = x3p[1] + a3 * t[:, 128:]
    W4v = W4p[...]
    Wr4v = Wr4[...]
    h4[...] = _dot(x4_0, W4v[:128, :]) + _dot(x4_1, W4v[128:, :])
    r4[...] = _dot(x4_0, Wr4v[:128, :]) + _dot(x4_1, Wr4v[128:, :]) + b4[...]


def _tc6_body(s4, invc, r4, x5):
    agg = (s4[0] + s4[1])[:, :112] * invc[...]
    x5[...] = agg + r4[...]


def _f32(*shape):
    return jax.ShapeDtypeStruct(shape, jnp.float32)


def kernel(x, edge_index, alpha, W0, b0, W1, Wr1, b1, W2, Wr2, b2,
           W3, b3, W4, Wr4, b4):
    npad = EPAD - E
    row = jnp.concatenate(
        [edge_index[0].astype(jnp.int32),
         jnp.full((npad,), DUMMY, jnp.int32)]).reshape(NCHUNKS, CHUNK)
    col = jnp.concatenate(
        [edge_index[1].astype(jnp.int32),
         jnp.zeros((npad,), jnp.int32)]).reshape(NCHUNKS, CHUNK)
    colp = jnp.concatenate([col, col + N], axis=0)
    zeros_d = jnp.zeros((NB, D), jnp.float32)
    ones_d = jnp.ones((CHUNK, D), jnp.float32)
    al = alpha.reshape(1, 5)
    b0r = b0.reshape(1, 128)
    b1r = b1.reshape(1, 256)
    b2r = b2.reshape(1, 256)
    b3r = b3.reshape(1, 256)
    b4r = b4.reshape(1, 112)
    W4p = jnp.pad(W4, ((0, 0), (0, 16)))

    deg2 = _deg(row, ones_d, zeros_d).reshape(2, ACC_R, 128)

    xs, dis, invc = pl.pallas_call(
        _tc1_body, grid=GRID,
        in_specs=[_b3(128), _b2(128)],
        out_specs=[_b2(128), _b2(1), _b2(1)],
        out_shape=[_f32(N, 128), _f32(N, 1), _f32(N, 1)],
    )(deg2, x)

    s0 = _seg_edges(xs, row, col, zeros_d).reshape(2, ACC_R, 128)

    x1 = pl.pallas_call(
        _tc2_body, grid=GRID,
        in_specs=[_b2(128), _b3(128), _b2(1), _bfull(128, 128),
                  _bfull(1, 128), _bfull(1, 5)],
        out_specs=_b2(128),
        out_shape=_f32(N, 128),
    )(x, s0, dis, W0, b0r, al)

    s1 = _seg_edges(x1, row, col, zeros_d).reshape(2, ACC_R, 128)

    x2p = pl.pallas_call(
        _tc3_body, grid=GRID,
        in_specs=[_b3(128), _b2(1), _b2(128), _bfull(128, 256),
                  _bfull(128, 256), _bfull(1, 256)],
        out_specs=_b3(128),
        out_shape=_f32(2, N, 128),
    )(s1, invc, x1, W1, Wr1, b1r)

    s2 = _seg_planes(x2p.reshape(2 * N, 128), row, colp,
                     zeros_d).reshape(2, ACC_R, 128)

    x3p, x3sp = pl.pallas_call(
        _tc4_body, grid=GRID,
        in_specs=[_b3(128), _b2(1), _b2(1), _b3(128), _bfull(256, 256),
                  _bfull(256, 256), _bfull(1, 256)],
        out_specs=[_b3(128), _b3(128)],
        out_shape=[_f32(2, N, 128), _f32(2, N, 128)],
    )(s2, invc, dis, x2p, W2, Wr2, b2r)

    s3 = _seg_planes(x3sp.reshape(2 * N, 128), row, colp,
                     zeros_d).reshape(2, ACC_R, 128)

    h4, r4 = pl.pallas_call(
        _tc5_body, grid=GRID,
        in_specs=[_b3(128), _b2(1), _b3(128), _bfull(256, 256),
                  _bfull(1, 256), _bfull(256, 128), _bfull(256, 112),
                  _bfull(1, 112), _bfull(1, 5)],
        out_specs=[_b2(128), _b2(112)],
        out_shape=[_f32(N, 128), _f32(N, 112)],
    )(s3, dis, x3p, W3, b3r, W4p, Wr4, b4r, al)

    s4 = _seg_edges(h4, row, col, zeros_d).reshape(2, ACC_R, 128)

    x5 = pl.pallas_call(
        _tc6_body, grid=GRID,
        in_specs=[_b3(128), _b2(1), _b2(112)],
        out_specs=_b2(112),
        out_shape=_f32(N, 112),
    )(s4, invc, r4)

    return x5
